# baseline (speedup 1.0000x reference)
import functools

import jax
import jax.numpy as jnp
from jax import lax
from jax.experimental import pallas as pl
from jax.experimental.pallas import tpu as pltpu
from jax.experimental.pallas import tpu_sc as plsc

_EMB = 128
_BATCH = 16384

_INFO = plsc.get_sparse_core_info()
_NC = _INFO.num_cores
_NS = _INFO.num_subcores
_NW = _NC * _NS
_BPW = _BATCH // _NW
_CHUNK = 128
_NCHUNK = _BPW // _CHUNK

_mesh = plsc.VectorSubcoreMesh(core_axis_name="c", subcore_axis_name="s")


@functools.partial(
    pl.kernel,
    mesh=_mesh,
    out_type=jax.ShapeDtypeStruct((_BATCH, _EMB), jnp.float32),
    scratch_types=[
        pltpu.VMEM((_NCHUNK, _CHUNK), jnp.int32),
        pltpu.VMEM((_BPW, _EMB), jnp.float32),
        pltpu.SemaphoreType.DMA,
    ],
)
def _embed_lookup(idx_hbm, table_hbm, out_hbm, idx_v, rows_v, sem):
    wid = lax.axis_index("s") * _NC + lax.axis_index("c")
    base = wid * _BPW
    pltpu.sync_copy(idx_hbm.at[pl.ds(wid * _NCHUNK, _NCHUNK)], idx_v)
    copies = [
        pltpu.async_copy(
            table_hbm.at[idx_v.at[j]],
            rows_v.at[pl.ds(j * _CHUNK, _CHUNK)],
            sem,
        )
        for j in range(_NCHUNK)
    ]
    for c in copies:
        c.wait()
    pltpu.sync_copy(rows_v, out_hbm.at[pl.ds(base, _BPW)])


def kernel(x, table):
    idx = x.reshape(_NW * _NCHUNK, _CHUNK).astype(jnp.int32)
    return _embed_lookup(idx, table)


# speedup vs baseline: 2.4009x; 2.4009x over previous
import functools

import jax
import jax.numpy as jnp
from jax import lax
from jax.experimental import pallas as pl
from jax.experimental.pallas import tpu as pltpu
from jax.experimental.pallas import tpu_sc as plsc

_EMB = 128
_BATCH = 16384
_VOCAB = 3

_INFO = plsc.get_sparse_core_info()
_NC = _INFO.num_cores
_NS = _INFO.num_subcores
_NW = _NC * _NS
_BPW = _BATCH // _NW
_L = _INFO.num_lanes
_NGROUP = _BPW // _L

_mesh = plsc.VectorSubcoreMesh(core_axis_name="c", subcore_axis_name="s")


@functools.partial(
    pl.kernel,
    mesh=_mesh,
    compiler_params=pltpu.CompilerParams(needs_layout_passes=False),
    out_type=jax.ShapeDtypeStruct((_BATCH * _EMB,), jnp.float32),
    scratch_types=[
        pltpu.VMEM((_BPW,), jnp.int32),
        pltpu.VMEM((_VOCAB * _EMB,), jnp.float32),
        pltpu.VMEM((_BPW * _EMB,), jnp.float32),
    ],
)
def _embed_lookup(idx_hbm, table_hbm, out_hbm, idx_v, table_v, out_v):
    wid = lax.axis_index("s") * _NC + lax.axis_index("c")
    pltpu.sync_copy(idx_hbm.at[wid], idx_v)
    pltpu.sync_copy(table_hbm, table_v)
    lane = lax.iota(jnp.int32, _L)

    def group(g, carry):
        vb = idx_v[pl.ds(g * _L, _L)]
        vsrc = vb * _EMB
        vdst = lane * _EMB + g * (_L * _EMB)
        for d in range(_EMB):
            val = plsc.load_gather(table_v, [vsrc + d])
            plsc.store_scatter(out_v, [vdst + d], val)
        return carry

    lax.fori_loop(0, _NGROUP, group, 0)
    pltpu.sync_copy(out_v, out_hbm.at[pl.ds(wid * _BPW * _EMB, _BPW * _EMB)])


def kernel(x, table):
    idx = x.reshape(_NW, _BPW).astype(jnp.int32)
    flat = _embed_lookup(idx, table.reshape(_VOCAB * _EMB))
    return flat.reshape(_BATCH, _EMB)


# speedup vs baseline: 9.0909x; 3.7864x over previous
import functools

import jax
import jax.numpy as jnp
from jax import lax
from jax.experimental import pallas as pl
from jax.experimental.pallas import tpu as pltpu
from jax.experimental.pallas import tpu_sc as plsc

_EMB = 128
_BATCH = 16384
_VOCAB = 3

_INFO = plsc.get_sparse_core_info()
_NC = _INFO.num_cores
_NS = _INFO.num_subcores
_NW = _NC * _NS
_BPW = _BATCH // _NW
_L = _INFO.num_lanes
_NCHW = _EMB // _L

_mesh = plsc.VectorSubcoreMesh(core_axis_name="c", subcore_axis_name="s")


@functools.partial(
    pl.kernel,
    mesh=_mesh,
    compiler_params=pltpu.CompilerParams(needs_layout_passes=False),
    out_type=jax.ShapeDtypeStruct((_BATCH * _EMB,), jnp.float32),
    scratch_types=[
        pltpu.VMEM((_BPW,), jnp.int32),
        pltpu.VMEM((_VOCAB * _EMB,), jnp.float32),
        pltpu.VMEM((_BPW * _EMB,), jnp.float32),
    ],
)
def _embed_lookup(idx_hbm, table_hbm, out_hbm, idx_v, table_v, out_v):
    wid = lax.axis_index("s") * _NC + lax.axis_index("c")
    pltpu.sync_copy(idx_hbm.at[wid], idx_v)
    pltpu.sync_copy(table_hbm, table_v)

    rows = [
        [table_v[pl.ds(k * _EMB + c * _L, _L)] for c in range(_NCHW)]
        for k in range(_VOCAB)
    ]

    @plsc.parallel_loop(0, _BPW, unroll=4)
    def _body(b):
        vidx = plsc.load_gather(idx_v, [jnp.full((_L,), 0, jnp.int32) + b])
        m0 = vidx == 0
        m1 = vidx == 1
        base = b * _EMB
        for c in range(_NCHW):
            val = jnp.where(m0, rows[0][c], jnp.where(m1, rows[1][c], rows[2][c]))
            out_v[pl.ds(base + c * _L, _L)] = val

    pltpu.sync_copy(out_v, out_hbm.at[pl.ds(wid * _BPW * _EMB, _BPW * _EMB)])


def kernel(x, table):
    idx = x.reshape(_NW, _BPW).astype(jnp.int32)
    flat = _embed_lookup(idx, table.reshape(_VOCAB * _EMB))
    return flat.reshape(_BATCH, _EMB)


# speedup vs baseline: 9.2170x; 1.0139x over previous
import functools

import jax
import jax.numpy as jnp
from jax import lax
from jax.experimental import pallas as pl
from jax.experimental.pallas import tpu as pltpu
from jax.experimental.pallas import tpu_sc as plsc

_EMB = 128
_BATCH = 16384
_VOCAB = 3

_INFO = plsc.get_sparse_core_info()
_NC = _INFO.num_cores
_NS = _INFO.num_subcores
_NW = _NC * _NS
_BPW = _BATCH // _NW
_L = _INFO.num_lanes
_NCHW = _EMB // _L
_GSZ = _L * _EMB
_NGRP = _BPW // _L

_mesh = plsc.VectorSubcoreMesh(core_axis_name="c", subcore_axis_name="s")


@functools.partial(
    pl.kernel,
    mesh=_mesh,
    compiler_params=pltpu.CompilerParams(needs_layout_passes=False),
    out_type=jax.ShapeDtypeStruct((_BATCH * _EMB,), jnp.float32),
    scratch_types=[
        pltpu.VMEM((_BPW,), jnp.int32),
        pltpu.VMEM((_VOCAB * _EMB,), jnp.float32),
        pltpu.VMEM((_BPW * _EMB,), jnp.float32),
        pltpu.SemaphoreType.DMA,
        pltpu.SemaphoreType.DMA,
    ],
)
def _embed_lookup(idx_hbm, table_hbm, out_hbm, idx_v, table_v, out_v, sem_in, sem_out):
    wid = lax.axis_index("s") * _NC + lax.axis_index("c")
    cp_idx = pltpu.async_copy(idx_hbm.at[wid], idx_v, sem_in)
    cp_tab = pltpu.async_copy(table_hbm, table_v, sem_in)
    cp_idx.wait()
    cp_tab.wait()

    rows = [
        [table_v[pl.ds(k * _EMB + c * _L, _L)] for c in range(_NCHW)]
        for k in range(_VOCAB)
    ]
    out_base = wid * (_BPW * _EMB)

    @plsc.parallel_loop(0, _NGRP, unroll=1)
    def _group(g):
        vidx = idx_v[pl.ds(g * _L, _L)]
        gbase = g * _GSZ
        for j in range(_L):
            vb = jnp.take_along_axis(
                vidx, jnp.full((_L,), j, jnp.int32), axis=0,
                mode="promise_in_bounds",
            )
            m0 = vb == 0
            m1 = vb == 1
            base = gbase + j * _EMB
            for c in range(_NCHW):
                val = jnp.where(m0, rows[0][c],
                                jnp.where(m1, rows[1][c], rows[2][c]))
                out_v[pl.ds(base + c * _L, _L)] = val
        pltpu.async_copy(
            out_v.at[pl.ds(gbase, _GSZ)],
            out_hbm.at[pl.ds(out_base + gbase, _GSZ)],
            sem_out,
        )

    pltpu.make_async_copy(
        out_hbm.at[pl.ds(out_base, _BPW * _EMB)], out_v, sem_out
    ).wait()


def kernel(x, table):
    idx = x.reshape(_NW, _BPW).astype(jnp.int32)
    flat = _embed_lookup(idx, table.reshape(_VOCAB * _EMB))
    return flat.reshape(_BATCH, _EMB)
